# baseline (speedup 1.0000x reference)
import functools

import jax
import jax.numpy as jnp
from jax import lax
from jax.experimental import pallas as pl
from jax.experimental.pallas import tpu as pltpu
from jax.experimental.pallas import tpu_sc as plsc

VOCAB = 100000
D = 512
B = 1024
L = 50

_NC = 2
_NS = 16
_NW = _NC * _NS

_N_ROWS = B * L
_N_OUT = B * (L + 1)
_NCHUNK = 4
_CROWS = _N_ROWS // _NCHUNK
_ROWS_PER_W = _CROWS // _NW
_CHUNK = 80
_N_CHUNKS = _ROWS_PER_W // _CHUNK


def _sc_gather_chunk(table, ids_chunk):
    mesh = plsc.VectorSubcoreMesh(core_axis_name="c", subcore_axis_name="s")

    @functools.partial(
        pl.kernel,
        mesh=mesh,
        out_type=jax.ShapeDtypeStruct((_CROWS, D), jnp.float32),
        scratch_types=[
            pltpu.VMEM((_ROWS_PER_W,), jnp.int32),
            pltpu.VMEM((3, _CHUNK, D), jnp.float32),
            pltpu.SemaphoreType.DMA,
            pltpu.SemaphoreType.DMA,
            pltpu.SemaphoreType.DMA,
            pltpu.SemaphoreType.DMA,
            pltpu.SemaphoreType.DMA,
            pltpu.SemaphoreType.DMA,
        ],
    )
    def gather_kernel(table_hbm, ids_hbm, out_hbm, idx_v, rows_v,
                      g0, g1, g2, p0, p1, p2):
        wid = lax.axis_index("s") * _NC + lax.axis_index("c")
        base = wid * _ROWS_PER_W
        gsem = (g0, g1, g2)
        psem = (p0, p1, p2)
        pltpu.sync_copy(ids_hbm.at[pl.ds(base, _ROWS_PER_W)], idx_v)

        def gather(c, buf):
            return pltpu.async_copy(
                table_hbm.at[idx_v.at[pl.ds(c * _CHUNK, _CHUNK)]],
                rows_v.at[buf], gsem[buf])

        gaths = [None, None, None]
        puts = [None, None, None]
        gaths[0] = gather(0, 0)
        if _N_CHUNKS > 1:
            gaths[1] = gather(1, 1)
        for c in range(_N_CHUNKS):
            b = c % 3
            if c + 2 < _N_CHUNKS:
                nb = (c + 2) % 3
                if puts[nb] is not None:
                    puts[nb].wait()
                    puts[nb] = None
                gaths[nb] = gather(c + 2, nb)
            gaths[b].wait()
            gaths[b] = None
            puts[b] = pltpu.async_copy(
                rows_v.at[b],
                out_hbm.at[pl.ds(base + c * _CHUNK, _CHUNK)],
                psem[b])
        for p in puts:
            if p is not None:
                p.wait()

    return gather_kernel(table, ids_chunk)


def _tc_img_init(img2d):
    def body(img_ref, o_ref):
        o_ref[...] = img_ref[...]

    return pl.pallas_call(
        body,
        grid=(1,),
        in_specs=[pl.BlockSpec((B, D), lambda r: (0, 0))],
        out_specs=pl.BlockSpec((B, D), lambda r: (L, 0)),
        out_shape=jax.ShapeDtypeStruct((_N_OUT, D), jnp.float32),
    )(img2d)


_RB = 3200
_TC_STEPS = _CROWS // _RB


def _tc_matmul_chunk(buf, rows_chunk, W1, i):
    def body(_, e_ref, w_ref, o_ref):
        o_ref[...] = jnp.dot(e_ref[...], w_ref[...],
                             preferred_element_type=jnp.float32)

    blk0 = i * _TC_STEPS
    return pl.pallas_call(
        body,
        grid=(_TC_STEPS,),
        in_specs=[
            pl.BlockSpec(memory_space=pltpu.MemorySpace.HBM),
            pl.BlockSpec((_RB, D), lambda r: (r, 0)),
            pl.BlockSpec((D, D), lambda r: (0, 0)),
        ],
        out_specs=pl.BlockSpec((_RB, D), lambda r: (blk0 + r, 0)),
        out_shape=jax.ShapeDtypeStruct((_N_OUT, D), jnp.float32),
        input_output_aliases={0: 0},
    )(buf, rows_chunk, W1)


def kernel(caption_tknID, img_fc, table0, W1):
    ids = caption_tknID.astype(jnp.int32).T.reshape(_N_ROWS)
    img2d = img_fc.reshape(B, D)
    chunks = [_sc_gather_chunk(table0, ids[i * _CROWS:(i + 1) * _CROWS])
              for i in range(_NCHUNK)]
    buf = _tc_img_init(img2d)
    for i in range(_NCHUNK):
        buf = _tc_matmul_chunk(buf, chunks[i], W1, i)
    return buf.reshape(L + 1, B, D).transpose(1, 0, 2)


# speedup vs baseline: 1.0144x; 1.0144x over previous
import functools

import jax
import jax.numpy as jnp
from jax import lax
from jax.experimental import pallas as pl
from jax.experimental.pallas import tpu as pltpu
from jax.experimental.pallas import tpu_sc as plsc

VOCAB = 100000
D = 512
B = 1024
L = 50

_NC = 2
_NS = 16
_NW = _NC * _NS

_N_ROWS = B * L
_N_OUT = B * (L + 1)
_NCHUNK = 2
_CROWS = _N_ROWS // _NCHUNK
_ROWS_PER_W = _CROWS // _NW
_CHUNK = 80
_N_CHUNKS = _ROWS_PER_W // _CHUNK


def _sc_gather_chunk(table, ids_chunk):
    mesh = plsc.VectorSubcoreMesh(core_axis_name="c", subcore_axis_name="s")

    @functools.partial(
        pl.kernel,
        mesh=mesh,
        out_type=jax.ShapeDtypeStruct((_CROWS, D), jnp.float32),
        scratch_types=[
            pltpu.VMEM((_ROWS_PER_W,), jnp.int32),
            pltpu.VMEM((3, _CHUNK, D), jnp.float32),
            pltpu.SemaphoreType.DMA,
            pltpu.SemaphoreType.DMA,
            pltpu.SemaphoreType.DMA,
            pltpu.SemaphoreType.DMA,
            pltpu.SemaphoreType.DMA,
            pltpu.SemaphoreType.DMA,
        ],
    )
    def gather_kernel(table_hbm, ids_hbm, out_hbm, idx_v, rows_v,
                      g0, g1, g2, p0, p1, p2):
        wid = lax.axis_index("s") * _NC + lax.axis_index("c")
        base = wid * _ROWS_PER_W
        gsem = (g0, g1, g2)
        psem = (p0, p1, p2)
        pltpu.sync_copy(ids_hbm.at[pl.ds(base, _ROWS_PER_W)], idx_v)

        def gather(c, buf):
            return pltpu.async_copy(
                table_hbm.at[idx_v.at[pl.ds(c * _CHUNK, _CHUNK)]],
                rows_v.at[buf], gsem[buf])

        gaths = [None, None, None]
        puts = [None, None, None]
        gaths[0] = gather(0, 0)
        if _N_CHUNKS > 1:
            gaths[1] = gather(1, 1)
        for c in range(_N_CHUNKS):
            b = c % 3
            if c + 2 < _N_CHUNKS:
                nb = (c + 2) % 3
                if puts[nb] is not None:
                    puts[nb].wait()
                    puts[nb] = None
                gaths[nb] = gather(c + 2, nb)
            gaths[b].wait()
            gaths[b] = None
            puts[b] = pltpu.async_copy(
                rows_v.at[b],
                out_hbm.at[pl.ds(base + c * _CHUNK, _CHUNK)],
                psem[b])
        for p in puts:
            if p is not None:
                p.wait()

    return gather_kernel(table, ids_chunk)


def _tc_img_init(img2d):
    def body(img_ref, o_ref):
        o_ref[...] = img_ref[...]

    return pl.pallas_call(
        body,
        grid=(1,),
        in_specs=[pl.BlockSpec((B, D), lambda r: (0, 0))],
        out_specs=pl.BlockSpec((B, D), lambda r: (L, 0)),
        out_shape=jax.ShapeDtypeStruct((_N_OUT, D), jnp.float32),
    )(img2d)


_RB = 3200
_TC_STEPS = _CROWS // _RB


def _tc_matmul_chunk(buf, rows_chunk, W1, i):
    def body(_, e_ref, w_ref, o_ref):
        o_ref[...] = jnp.dot(e_ref[...], w_ref[...],
                             preferred_element_type=jnp.float32)

    blk0 = i * _TC_STEPS
    return pl.pallas_call(
        body,
        grid=(_TC_STEPS,),
        in_specs=[
            pl.BlockSpec(memory_space=pltpu.MemorySpace.HBM),
            pl.BlockSpec((_RB, D), lambda r: (r, 0)),
            pl.BlockSpec((D, D), lambda r: (0, 0)),
        ],
        out_specs=pl.BlockSpec((_RB, D), lambda r: (blk0 + r, 0)),
        out_shape=jax.ShapeDtypeStruct((_N_OUT, D), jnp.float32),
        input_output_aliases={0: 0},
    )(buf, rows_chunk, W1)


def kernel(caption_tknID, img_fc, table0, W1):
    ids = caption_tknID.astype(jnp.int32).T.reshape(_N_ROWS)
    img2d = img_fc.reshape(B, D)
    chunks = [_sc_gather_chunk(table0, ids[i * _CROWS:(i + 1) * _CROWS])
              for i in range(_NCHUNK)]
    buf = _tc_img_init(img2d)
    for i in range(_NCHUNK):
        buf = _tc_matmul_chunk(buf, chunks[i], W1, i)
    return buf.reshape(L + 1, B, D).transpose(1, 0, 2)


# speedup vs baseline: 1.0211x; 1.0067x over previous
import functools

import jax
import jax.numpy as jnp
from jax import lax
from jax.experimental import pallas as pl
from jax.experimental.pallas import tpu as pltpu
from jax.experimental.pallas import tpu_sc as plsc

VOCAB = 100000
D = 512
B = 1024
L = 50

_NC = 2
_NS = 16
_NW = _NC * _NS

_N_ROWS = B * L
_N_OUT = B * (L + 1)
_ROWS_PER_W = _N_ROWS // _NW
_IMG_PER_W = B // _NW
_CHUNK = 80
_N_CHUNKS = _ROWS_PER_W // _CHUNK


def _sc_gather_concat(table, ids, img2d):
    mesh = plsc.VectorSubcoreMesh(core_axis_name="c", subcore_axis_name="s")

    @functools.partial(
        pl.kernel,
        mesh=mesh,
        out_type=jax.ShapeDtypeStruct((_N_OUT, D), jnp.float32),
        scratch_types=[
            pltpu.VMEM((_ROWS_PER_W,), jnp.int32),
            pltpu.VMEM((3, _CHUNK, D), jnp.float32),
            pltpu.SemaphoreType.DMA,
            pltpu.SemaphoreType.DMA,
            pltpu.SemaphoreType.DMA,
            pltpu.SemaphoreType.DMA,
            pltpu.SemaphoreType.DMA,
            pltpu.SemaphoreType.DMA,
        ],
    )
    def gather_kernel(table_hbm, ids_hbm, img_hbm, out_hbm, idx_v, rows_v,
                      g0, g1, g2, p0, p1, p2):
        wid = lax.axis_index("s") * _NC + lax.axis_index("c")
        base = wid * _ROWS_PER_W
        gsem = (g0, g1, g2)
        psem = (p0, p1, p2)
        pltpu.sync_copy(ids_hbm.at[pl.ds(base, _ROWS_PER_W)], idx_v)
        pltpu.sync_copy(img_hbm.at[pl.ds(wid * _IMG_PER_W, _IMG_PER_W)],
                        rows_v.at[0, pl.ds(0, _IMG_PER_W)])
        pltpu.sync_copy(rows_v.at[0, pl.ds(0, _IMG_PER_W)],
                        out_hbm.at[pl.ds(_N_ROWS + wid * _IMG_PER_W,
                                         _IMG_PER_W)])

        def gather(c, buf):
            return pltpu.async_copy(
                table_hbm.at[idx_v.at[pl.ds(c * _CHUNK, _CHUNK)]],
                rows_v.at[buf], gsem[buf])

        gaths = [None, None, None]
        puts = [None, None, None]
        gaths[0] = gather(0, 0)
        if _N_CHUNKS > 1:
            gaths[1] = gather(1, 1)
        for c in range(_N_CHUNKS):
            b = c % 3
            if c + 2 < _N_CHUNKS:
                nb = (c + 2) % 3
                if puts[nb] is not None:
                    puts[nb].wait()
                    puts[nb] = None
                gaths[nb] = gather(c + 2, nb)
            gaths[b].wait()
            gaths[b] = None
            puts[b] = pltpu.async_copy(
                rows_v.at[b],
                out_hbm.at[pl.ds(base + c * _CHUNK, _CHUNK)],
                psem[b])
        for p in puts:
            if p is not None:
                p.wait()

    return gather_kernel(table, ids, img2d)


_RB = 3072
_N_TC = _N_OUT // _RB


def _tc_matmul(rows2d, W1):
    def body(e_ref, w_ref, o_ref):
        r = pl.program_id(0)

        @pl.when(r < _N_TC - 1)
        def _():
            o_ref[...] = jnp.dot(e_ref[...], w_ref[...],
                                 preferred_element_type=jnp.float32)

        @pl.when(r == _N_TC - 1)
        def _():
            split = _N_ROWS - (_N_TC - 1) * _RB
            o_ref[:split, :] = jnp.dot(e_ref[:split, :], w_ref[...],
                                       preferred_element_type=jnp.float32)
            o_ref[split:, :] = e_ref[split:, :]

    return pl.pallas_call(
        body,
        grid=(_N_TC,),
        in_specs=[
            pl.BlockSpec((_RB, D), lambda r: (r, 0)),
            pl.BlockSpec((D, D), lambda r: (0, 0)),
        ],
        out_specs=pl.BlockSpec((_RB, D), lambda r: (r, 0)),
        out_shape=jax.ShapeDtypeStruct((_N_OUT, D), jnp.float32),
    )(rows2d, W1)


def kernel(caption_tknID, img_fc, table0, W1):
    ids = caption_tknID.astype(jnp.int32).T.reshape(_N_ROWS)
    img2d = img_fc.reshape(B, D)
    rows = _sc_gather_concat(table0, ids, img2d)
    out2d = _tc_matmul(rows, W1)
    return out2d.reshape(L + 1, B, D).transpose(1, 0, 2)
